# baseline (speedup 1.0000x reference)
import functools

import jax
import jax.numpy as jnp
from jax import lax
from jax.experimental import pallas as pl
from jax.experimental.pallas import tpu as pltpu
from jax.experimental.pallas import tpu_sc as plsc

N = 10000
M = 16
A = 256
NB = 128
S = 128
B = 128

BN = 200
CH = 1
NCH = N // CH
GRIDC = NCH // BN
NPC = 10240

_NC = 2
_NS = 16
_NW = _NC * _NS


def _sigmoid(x):
    return 0.5 * jnp.tanh(0.5 * x) + 0.5


def _softplus(x):
    return jnp.maximum(x, 0.0) + jnp.log1p(jnp.exp(-jnp.abs(x)))


_CHUNK = 40
_K = 5


def _gather_loop(table_hbm, out_hbm, idx_v, bufs, sem, rpw, wbase):
    ngroups = rpw // (_K * _CHUNK)

    def group(g, carry):
        goff = g * (_K * _CHUNK)
        copies = []
        for j in range(_K):
            copies.append(pltpu.async_copy(
                table_hbm.at[idx_v.at[pl.ds(goff + j * _CHUNK, _CHUNK)]],
                bufs[j], sem))
        for j in range(_K):
            copies[j].wait()
            pltpu.sync_copy(
                bufs[j], out_hbm.at[pl.ds(wbase + goff + j * _CHUNK, _CHUNK)])
        return carry

    lax.fori_loop(0, ngroups, group, 0)


def _gather_body(rpw, srpw, table_hbm, idx_hbm, st_hbm, sidx_hbm,
                 out_hbm, sout_hbm, idx_v, sidx_v, sbuf, *rest):
    bufs, sem = rest[:_K], rest[_K]
    wid = lax.axis_index("s") * _NC + lax.axis_index("c")

    base = wid * rpw
    pltpu.sync_copy(idx_hbm.at[pl.ds(base, rpw)], idx_v)
    _gather_loop(table_hbm, out_hbm, idx_v, bufs, sem, rpw, base)

    sbase = wid * srpw
    pltpu.sync_copy(sidx_hbm.at[pl.ds(sbase, srpw)], sidx_v)

    def sstep(c, carry):
        off = c * _CHUNK
        pltpu.async_copy(
            st_hbm.at[sidx_v.at[pl.ds(off, _CHUNK)]], sbuf, sem).wait()
        pltpu.sync_copy(sbuf, sout_hbm.at[pl.ds(sbase + off, _CHUNK)])
        return carry

    lax.fori_loop(0, srpw // _CHUNK, sstep, 0)


def _gather2_body(rpw, table_hbm, idx_hbm, out_hbm, idx_v, *rest):
    bufs, sem = rest[:_K], rest[_K]
    wid = lax.axis_index("s") * _NC + lax.axis_index("c")
    base = wid * rpw
    pltpu.sync_copy(idx_hbm.at[pl.ds(base, rpw)], idx_v)
    _gather_loop(table_hbm, out_hbm, idx_v, bufs, sem, rpw, base)


def _mesh():
    return plsc.VectorSubcoreMesh(
        core_axis_name="c", subcore_axis_name="s",
        num_cores=_NC, num_subcores=_NS)


@functools.cache
def _make_gather_st(R, NP):
    rpw = R // _NW
    srpw = NP // _NW
    return pl.kernel(
        functools.partial(_gather_body, rpw, srpw),
        out_type=(jax.ShapeDtypeStruct((R, 128), jnp.int32),
                  jax.ShapeDtypeStruct((NP, S), jnp.float32)),
        mesh=_mesh(),
        scratch_types=[
            pltpu.VMEM((rpw,), jnp.int32),
            pltpu.VMEM((srpw,), jnp.int32),
            pltpu.VMEM((_CHUNK, S), jnp.float32),
        ] + [pltpu.VMEM((_CHUNK, 128), jnp.int32) for _ in range(_K)]
          + [pltpu.SemaphoreType.DMA],
    )


@functools.cache
def _make_gather(R):
    rpw = R // _NW
    return pl.kernel(
        functools.partial(_gather2_body, rpw),
        out_type=jax.ShapeDtypeStruct((R, 128), jnp.int32),
        mesh=_mesh(),
        scratch_types=[
            pltpu.VMEM((rpw,), jnp.int32),
        ] + [pltpu.VMEM((_CHUNK, 128), jnp.int32) for _ in range(_K)]
          + [pltpu.SemaphoreType.DMA],
    )


def _atom_stage(a_ref, g_ref, nbr_ref, st_ref, wself, wnbr, wbond, wst, ba_ref,
                out_ref):
    a = a_ref[...]
    base = (jnp.dot(a, wself[...], preferred_element_type=jnp.float32)
            + jnp.dot(st_ref[...], wst[...], preferred_element_type=jnp.float32)
            + ba_ref[...])
    g = g_ref[...].reshape(BN * M, A)
    nb = nbr_ref[...].reshape(BN * M, NB)
    zz = (jnp.dot(g, wnbr[...], preferred_element_type=jnp.float32)
          + jnp.dot(nb, wbond[...], preferred_element_type=jnp.float32))
    z = zz.reshape(BN, M, 2 * A) + base[:, None, :]
    filt = z[..., :A]
    core = z[..., A:]
    acc = jnp.sum(_sigmoid(filt) * _softplus(core), axis=1)
    out_ref[...] = _softplus(a + acc)


def _bond_stage(nbr_ref, g_ref, ao_ref, st_ref, idx_ref, wself, wnbr, wbond,
                wst, bb_ref, nbr_out_ref, pools_ref, acc_a, acc_b, acc_c):
    i = pl.program_id(0)

    @pl.when(i == 0)
    def _():
        acc_a[...] = jnp.zeros_like(acc_a)
        acc_b[...] = jnp.zeros_like(acc_b)
        acc_c[...] = jnp.zeros_like(acc_c)

    ao = ao_ref[...]
    ub = (jnp.dot(ao, wself[...], preferred_element_type=jnp.float32)
          + jnp.dot(st_ref[...], wst[...], preferred_element_type=jnp.float32)
          + bb_ref[...])
    g = g_ref[...].reshape(BN * M, A)
    nb3 = nbr_ref[...]
    nb = nb3.reshape(BN * M, NB).astype(jnp.bfloat16)
    zz = (jnp.dot(g, wnbr[...], preferred_element_type=jnp.float32)
          + jnp.dot(nb, wbond[...], preferred_element_type=jnp.float32))
    z = zz.reshape(BN, M, 2 * NB) + ub[:, None, :]
    filt = z[..., :NB]
    core = z[..., NB:]
    no = _softplus(nb3 + _sigmoid(filt) * _softplus(core))
    nbr_out_ref[...] = no

    bm = jnp.mean(no, axis=1)
    idx_t = idx_ref[...].reshape(1, BN)
    oh = (lax.broadcasted_iota(jnp.int32, (B, BN), 0) == idx_t
          ).astype(jnp.float32)
    acc_a[...] = acc_a[...] + jnp.dot(oh, ao, preferred_element_type=jnp.float32)
    acc_b[...] = acc_b[...] + jnp.dot(oh, bm, preferred_element_type=jnp.float32)
    acc_c[...] = acc_c[...] + jnp.sum(oh, axis=1, keepdims=True)

    @pl.when(i == GRIDC - 1)
    def _():
        pools_ref[...] = jnp.concatenate(
            [acc_a[...], acc_b[...], acc_c[...]], axis=-1)


def _state_stage(pools_ref, sf_ref, ws_ref, bs_ref, out_ref):
    p = pools_ref[...]
    cnt = jnp.maximum(p[..., A + NB:], 1.0)
    cnt_a = jnp.concatenate([cnt, cnt], axis=-1)
    ap = p[..., :A] / cnt_a
    bp = p[..., A:A + NB] / cnt
    sf = sf_ref[...]
    t3 = jnp.concatenate([ap, bp, sf], axis=-1)
    out_ref[...] = _softplus(
        sf + jnp.dot(t3, ws_ref[...], preferred_element_type=jnp.float32)
        + bs_ref[...])


def _full(shape):
    nd = len(shape)
    return pl.BlockSpec(shape, lambda i: (0,) * nd)


def kernel(atom_fea, nbr_fea, state_fea, Wa, ba, Wb, bb, Ws, bs,
           atom_nbr_idx, node_atom_idx):
    flat_idx = atom_nbr_idx.reshape(-1).astype(jnp.int32)
    node_idx = node_atom_idx.astype(jnp.int32)

    wa_self, wa_nbr = Wa[:A], Wa[A:2 * A].astype(jnp.bfloat16)
    wa_bond, wa_st = Wa[2 * A:2 * A + NB].astype(jnp.bfloat16), Wa[2 * A + NB:]
    wb_self, wb_nbr = Wb[:A], Wb[A:2 * A].astype(jnp.bfloat16)
    wb_bond, wb_st = Wb[2 * A:2 * A + NB].astype(jnp.bfloat16), Wb[2 * A + NB:]
    ba2 = ba.reshape(1, 2 * A)
    bb2 = bb.reshape(1, 2 * NB)
    bs2 = bs.reshape(1, S)
    nbr_bf = nbr_fea.astype(jnp.bfloat16)
    table1 = lax.bitcast_convert_type(
        atom_fea.astype(jnp.bfloat16).reshape(N, 128, 2), jnp.int32)

    gather_st = _make_gather_st(N * M, NPC)
    gather = _make_gather(N * M)
    node_idx3 = node_idx.reshape(GRIDC, 1, BN)

    sidx = jnp.concatenate([node_idx, jnp.zeros((NPC - N,), jnp.int32)])
    g1i, st_full = gather_st(table1, flat_idx, state_fea, sidx)
    g1 = lax.bitcast_convert_type(g1i, jnp.bfloat16).reshape(N, M, A)
    st = st_full[:N]

    atom_out = pl.pallas_call(
        _atom_stage,
        grid=(GRIDC,),
        in_specs=[
            pl.BlockSpec((BN, A), lambda i: (i, 0)),
            pl.BlockSpec((BN, M, A), lambda i: (i, 0, 0)),
            pl.BlockSpec((BN, M, NB), lambda i: (i, 0, 0)),
            pl.BlockSpec((BN, S), lambda i: (i, 0)),
            _full((A, 2 * A)),
            _full((A, 2 * A)),
            _full((NB, 2 * A)),
            _full((S, 2 * A)),
            _full((1, 2 * A)),
        ],
        out_specs=pl.BlockSpec((BN, A), lambda i: (i, 0)),
        out_shape=jax.ShapeDtypeStruct((N, A), jnp.float32),
    )(atom_fea, g1, nbr_bf, st, wa_self, wa_nbr, wa_bond, wa_st, ba2)

    table2 = lax.bitcast_convert_type(
        atom_out.astype(jnp.bfloat16).reshape(N, 128, 2), jnp.int32)
    g2 = lax.bitcast_convert_type(
        gather(table2, flat_idx), jnp.bfloat16).reshape(N, M, A)

    nbr_out, pools = pl.pallas_call(
        _bond_stage,
        grid=(GRIDC,),
        in_specs=[
            pl.BlockSpec((BN, M, NB), lambda i: (i, 0, 0)),
            pl.BlockSpec((BN, M, A), lambda i: (i, 0, 0)),
            pl.BlockSpec((BN, A), lambda i: (i, 0)),
            pl.BlockSpec((BN, S), lambda i: (i, 0)),
            pl.BlockSpec((1, 1, BN), lambda i: (i, 0, 0)),
            _full((A, 2 * NB)),
            _full((A, 2 * NB)),
            _full((NB, 2 * NB)),
            _full((S, 2 * NB)),
            _full((1, 2 * NB)),
        ],
        out_specs=[
            pl.BlockSpec((BN, M, NB), lambda i: (i, 0, 0)),
            pl.BlockSpec((B, A + 2 * NB), lambda i: (0, 0)),
        ],
        out_shape=[
            jax.ShapeDtypeStruct((N, M, NB), jnp.float32),
            jax.ShapeDtypeStruct((B, A + 2 * NB), jnp.float32),
        ],
        scratch_shapes=[
            pltpu.VMEM((B, A), jnp.float32),
            pltpu.VMEM((B, NB), jnp.float32),
            pltpu.VMEM((B, NB), jnp.float32),
        ],
    )(nbr_fea, g2, atom_out, st, node_idx3,
      wb_self, wb_nbr, wb_bond, wb_st, bb2)

    state_out = pl.pallas_call(
        _state_stage,
        grid=(1,),
        in_specs=[
            _full((B, A + 2 * NB)),
            _full((B, S)),
            _full((A + NB + S, S)),
            _full((1, S)),
        ],
        out_specs=pl.BlockSpec((B, S), lambda i: (0, 0)),
        out_shape=jax.ShapeDtypeStruct((B, S), jnp.float32),
    )(pools, state_fea, Ws, bs2)

    return atom_out, nbr_out, state_out


# speedup vs baseline: 6.6499x; 6.6499x over previous
import functools

import jax
import jax.numpy as jnp
from jax import lax
from jax.experimental import pallas as pl
from jax.experimental.pallas import tpu as pltpu
from jax.experimental.pallas import tpu_sc as plsc

N = 10000
M = 16
A = 256
NB = 128
S = 128
B = 128

BN = 200
CH = 1
NCH = N // CH
GRIDC = NCH // BN
NPC = 10240

_NC = 2
_NS = 16
_NW = _NC * _NS


def _sigmoid(x):
    return 0.5 * jnp.tanh(0.5 * x) + 0.5


def _softplus(x):
    return jnp.maximum(x, 0.0) + jnp.log1p(jnp.exp(-jnp.abs(x)))


_CHUNK = 40
_K = 5


def _gather_loop(table_hbm, out_hbm, idx_v, bufs, sem, rpw, wbase):
    ngroups = rpw // (_K * _CHUNK)

    def group(g, carry):
        goff = g * (_K * _CHUNK)
        copies = []
        for j in range(_K):
            copies.append(pltpu.async_copy(
                table_hbm.at[idx_v.at[pl.ds(goff + j * _CHUNK, _CHUNK)]],
                bufs[j], sem))
        for j in range(_K):
            copies[j].wait()
            pltpu.sync_copy(
                bufs[j], out_hbm.at[pl.ds(wbase + goff + j * _CHUNK, _CHUNK)])
        return carry

    lax.fori_loop(0, ngroups, group, 0)


def _gather_body(rpw, srpw, table_hbm, idx_hbm, st_hbm, sidx_hbm,
                 out_hbm, sout_hbm, idx_v, sidx_v, sbuf, *rest):
    bufs, sem = rest[:_K], rest[_K]
    wid = lax.axis_index("s") * _NC + lax.axis_index("c")

    base = wid * rpw
    pltpu.sync_copy(idx_hbm.at[pl.ds(base, rpw)], idx_v)
    _gather_loop(table_hbm, out_hbm, idx_v, bufs, sem, rpw, base)

    sbase = wid * srpw
    pltpu.sync_copy(sidx_hbm.at[pl.ds(sbase, srpw)], sidx_v)

    def sstep(c, carry):
        off = c * _CHUNK
        pltpu.async_copy(
            st_hbm.at[sidx_v.at[pl.ds(off, _CHUNK)]], sbuf, sem).wait()
        pltpu.sync_copy(sbuf, sout_hbm.at[pl.ds(sbase + off, _CHUNK)])
        return carry

    lax.fori_loop(0, srpw // _CHUNK, sstep, 0)


def _gather2_body(rpw, table_hbm, idx_hbm, out_hbm, idx_v, *rest):
    bufs, sem = rest[:_K], rest[_K]
    wid = lax.axis_index("s") * _NC + lax.axis_index("c")
    base = wid * rpw
    pltpu.sync_copy(idx_hbm.at[pl.ds(base, rpw)], idx_v)
    _gather_loop(table_hbm, out_hbm, idx_v, bufs, sem, rpw, base)


def _mesh():
    return plsc.VectorSubcoreMesh(
        core_axis_name="c", subcore_axis_name="s",
        num_cores=_NC, num_subcores=_NS)


@functools.cache
def _make_gather_st(R, NP):
    rpw = R // _NW
    srpw = NP // _NW
    return pl.kernel(
        functools.partial(_gather_body, rpw, srpw),
        out_type=(jax.ShapeDtypeStruct((R, A), jnp.float32),
                  jax.ShapeDtypeStruct((NP, S), jnp.float32)),
        mesh=_mesh(),
        scratch_types=[
            pltpu.VMEM((rpw,), jnp.int32),
            pltpu.VMEM((srpw,), jnp.int32),
            pltpu.VMEM((_CHUNK, S), jnp.float32),
        ] + [pltpu.VMEM((_CHUNK, A), jnp.float32) for _ in range(_K)]
          + [pltpu.SemaphoreType.DMA],
    )


@functools.cache
def _make_gather(R):
    rpw = R // _NW
    return pl.kernel(
        functools.partial(_gather2_body, rpw),
        out_type=jax.ShapeDtypeStruct((R, A), jnp.float32),
        mesh=_mesh(),
        scratch_types=[
            pltpu.VMEM((rpw,), jnp.int32),
        ] + [pltpu.VMEM((_CHUNK, A), jnp.float32) for _ in range(_K)]
          + [pltpu.SemaphoreType.DMA],
    )


def _atom_stage(a_ref, g_ref, nbr_ref, st_ref, wself, wnbr, wbond, wst, ba_ref,
                out_ref):
    a = a_ref[...]
    base = (jnp.dot(a, wself[...], preferred_element_type=jnp.float32)
            + jnp.dot(st_ref[...], wst[...], preferred_element_type=jnp.float32)
            + ba_ref[...])
    g = g_ref[...].reshape(BN * M, A).astype(jnp.bfloat16)
    nb = nbr_ref[...].reshape(BN * M, NB)
    zz = (jnp.dot(g, wnbr[...], preferred_element_type=jnp.float32)
          + jnp.dot(nb, wbond[...], preferred_element_type=jnp.float32))
    z = zz.reshape(BN, M, 2 * A) + base[:, None, :]
    filt = z[..., :A]
    core = z[..., A:]
    acc = jnp.sum(_sigmoid(filt) * _softplus(core), axis=1)
    out_ref[...] = _softplus(a + acc)


def _bond_stage(nbr_ref, g_ref, ao_ref, st_ref, idx_ref, wself, wnbr, wbond,
                wst, bb_ref, nbr_out_ref, pools_ref, acc_a, acc_b, acc_c):
    i = pl.program_id(0)

    @pl.when(i == 0)
    def _():
        acc_a[...] = jnp.zeros_like(acc_a)
        acc_b[...] = jnp.zeros_like(acc_b)
        acc_c[...] = jnp.zeros_like(acc_c)

    ao = ao_ref[...]
    ub = (jnp.dot(ao, wself[...], preferred_element_type=jnp.float32)
          + jnp.dot(st_ref[...], wst[...], preferred_element_type=jnp.float32)
          + bb_ref[...])
    g = g_ref[...].reshape(BN * M, A).astype(jnp.bfloat16)
    nb3 = nbr_ref[...]
    nb = nb3.reshape(BN * M, NB).astype(jnp.bfloat16)
    zz = (jnp.dot(g, wnbr[...], preferred_element_type=jnp.float32)
          + jnp.dot(nb, wbond[...], preferred_element_type=jnp.float32))
    z = zz.reshape(BN, M, 2 * NB) + ub[:, None, :]
    filt = z[..., :NB]
    core = z[..., NB:]
    no = _softplus(nb3 + _sigmoid(filt) * _softplus(core))
    nbr_out_ref[...] = no

    bm = jnp.mean(no, axis=1)
    idx_t = idx_ref[...].reshape(1, BN)
    oh = (lax.broadcasted_iota(jnp.int32, (B, BN), 0) == idx_t
          ).astype(jnp.float32)
    acc_a[...] = acc_a[...] + jnp.dot(oh, ao, preferred_element_type=jnp.float32)
    acc_b[...] = acc_b[...] + jnp.dot(oh, bm, preferred_element_type=jnp.float32)
    acc_c[...] = acc_c[...] + jnp.sum(oh, axis=1, keepdims=True)

    @pl.when(i == GRIDC - 1)
    def _():
        pools_ref[...] = jnp.concatenate(
            [acc_a[...], acc_b[...], acc_c[...]], axis=-1)


def _state_stage(pools_ref, sf_ref, ws_ref, bs_ref, out_ref):
    p = pools_ref[...]
    cnt = jnp.maximum(p[..., A + NB:], 1.0)
    cnt_a = jnp.concatenate([cnt, cnt], axis=-1)
    ap = p[..., :A] / cnt_a
    bp = p[..., A:A + NB] / cnt
    sf = sf_ref[...]
    t3 = jnp.concatenate([ap, bp, sf], axis=-1)
    out_ref[...] = _softplus(
        sf + jnp.dot(t3, ws_ref[...], preferred_element_type=jnp.float32)
        + bs_ref[...])


def _full(shape):
    nd = len(shape)
    return pl.BlockSpec(shape, lambda i: (0,) * nd)


def kernel(atom_fea, nbr_fea, state_fea, Wa, ba, Wb, bb, Ws, bs,
           atom_nbr_idx, node_atom_idx):
    flat_idx = atom_nbr_idx.reshape(-1).astype(jnp.int32)
    node_idx = node_atom_idx.astype(jnp.int32)

    wa_self, wa_nbr = Wa[:A], Wa[A:2 * A].astype(jnp.bfloat16)
    wa_bond, wa_st = Wa[2 * A:2 * A + NB].astype(jnp.bfloat16), Wa[2 * A + NB:]
    wb_self, wb_nbr = Wb[:A], Wb[A:2 * A].astype(jnp.bfloat16)
    wb_bond, wb_st = Wb[2 * A:2 * A + NB].astype(jnp.bfloat16), Wb[2 * A + NB:]
    ba2 = ba.reshape(1, 2 * A)
    bb2 = bb.reshape(1, 2 * NB)
    bs2 = bs.reshape(1, S)
    nbr_bf = nbr_fea.astype(jnp.bfloat16)

    gather_st = _make_gather_st(N * M, NPC)
    gather = _make_gather(N * M)
    node_idx3 = node_idx.reshape(GRIDC, 1, BN)

    sidx = jnp.concatenate([node_idx, jnp.zeros((NPC - N,), jnp.int32)])
    g1, st_full = gather_st(atom_fea, flat_idx, state_fea, sidx)
    g1 = g1.reshape(N, M, A)
    st = st_full[:N]

    atom_out = pl.pallas_call(
        _atom_stage,
        grid=(GRIDC,),
        in_specs=[
            pl.BlockSpec((BN, A), lambda i: (i, 0)),
            pl.BlockSpec((BN, M, A), lambda i: (i, 0, 0)),
            pl.BlockSpec((BN, M, NB), lambda i: (i, 0, 0)),
            pl.BlockSpec((BN, S), lambda i: (i, 0)),
            _full((A, 2 * A)),
            _full((A, 2 * A)),
            _full((NB, 2 * A)),
            _full((S, 2 * A)),
            _full((1, 2 * A)),
        ],
        out_specs=pl.BlockSpec((BN, A), lambda i: (i, 0)),
        out_shape=jax.ShapeDtypeStruct((N, A), jnp.float32),
    )(atom_fea, g1, nbr_bf, st, wa_self, wa_nbr, wa_bond, wa_st, ba2)

    g2 = gather(atom_out, flat_idx).reshape(N, M, A)

    nbr_out, pools = pl.pallas_call(
        _bond_stage,
        grid=(GRIDC,),
        in_specs=[
            pl.BlockSpec((BN, M, NB), lambda i: (i, 0, 0)),
            pl.BlockSpec((BN, M, A), lambda i: (i, 0, 0)),
            pl.BlockSpec((BN, A), lambda i: (i, 0)),
            pl.BlockSpec((BN, S), lambda i: (i, 0)),
            pl.BlockSpec((1, 1, BN), lambda i: (i, 0, 0)),
            _full((A, 2 * NB)),
            _full((A, 2 * NB)),
            _full((NB, 2 * NB)),
            _full((S, 2 * NB)),
            _full((1, 2 * NB)),
        ],
        out_specs=[
            pl.BlockSpec((BN, M, NB), lambda i: (i, 0, 0)),
            pl.BlockSpec((B, A + 2 * NB), lambda i: (0, 0)),
        ],
        out_shape=[
            jax.ShapeDtypeStruct((N, M, NB), jnp.float32),
            jax.ShapeDtypeStruct((B, A + 2 * NB), jnp.float32),
        ],
        scratch_shapes=[
            pltpu.VMEM((B, A), jnp.float32),
            pltpu.VMEM((B, NB), jnp.float32),
            pltpu.VMEM((B, NB), jnp.float32),
        ],
    )(nbr_fea, g2, atom_out, st, node_idx3,
      wb_self, wb_nbr, wb_bond, wb_st, bb2)

    state_out = pl.pallas_call(
        _state_stage,
        grid=(1,),
        in_specs=[
            _full((B, A + 2 * NB)),
            _full((B, S)),
            _full((A + NB + S, S)),
            _full((1, S)),
        ],
        out_specs=pl.BlockSpec((B, S), lambda i: (0, 0)),
        out_shape=jax.ShapeDtypeStruct((B, S), jnp.float32),
    )(pools, state_fea, Ws, bs2)

    return atom_out, nbr_out, state_out


# speedup vs baseline: 6.6567x; 1.0010x over previous
import functools

import jax
import jax.numpy as jnp
from jax import lax
from jax.experimental import pallas as pl
from jax.experimental.pallas import tpu as pltpu
from jax.experimental.pallas import tpu_sc as plsc

N = 10000
M = 16
A = 256
NB = 128
S = 128
B = 128

BN = 200
CH = 1
NCH = N // CH
GRIDC = NCH // BN
NPC = 10240

_NC = 2
_NS = 16
_NW = _NC * _NS


def _sigmoid(x):
    return 0.5 * jnp.tanh(0.5 * x) + 0.5


def _softplus(x):
    return jnp.maximum(x, 0.0) + jnp.log1p(jnp.exp(-jnp.abs(x)))


_CHUNK = 40
_K = 5


def _gather_loop(table_hbm, out_hbm, idx_v, bufs, sem, rpw, wbase):
    ngroups = rpw // (_K * _CHUNK)

    def group(g, carry):
        goff = g * (_K * _CHUNK)
        copies = []
        for j in range(_K):
            copies.append(pltpu.async_copy(
                table_hbm.at[idx_v.at[pl.ds(goff + j * _CHUNK, _CHUNK)]],
                bufs[j], sem))
        for j in range(_K):
            copies[j].wait()
            pltpu.sync_copy(
                bufs[j], out_hbm.at[pl.ds(wbase + goff + j * _CHUNK, _CHUNK)])
        return carry

    lax.fori_loop(0, ngroups, group, 0)


def _gather_body(rpw, srpw, table_hbm, idx_hbm, st_hbm, sidx_hbm,
                 out_hbm, sout_hbm, idx_v, sidx_v, sbuf, *rest):
    bufs, sem = rest[:_K], rest[_K]
    wid = lax.axis_index("s") * _NC + lax.axis_index("c")

    base = wid * rpw
    pltpu.sync_copy(idx_hbm.at[pl.ds(base, rpw)], idx_v)
    _gather_loop(table_hbm, out_hbm, idx_v, bufs, sem, rpw, base)

    sbase = wid * srpw
    pltpu.sync_copy(sidx_hbm.at[pl.ds(sbase, srpw)], sidx_v)

    def sstep(c, carry):
        off = c * _CHUNK
        pltpu.async_copy(
            st_hbm.at[sidx_v.at[pl.ds(off, _CHUNK)]], sbuf, sem).wait()
        pltpu.sync_copy(sbuf, sout_hbm.at[pl.ds(sbase + off, _CHUNK)])
        return carry

    lax.fori_loop(0, srpw // _CHUNK, sstep, 0)


def _gather2_body(rpw, table_hbm, idx_hbm, out_hbm, idx_v, *rest):
    bufs, sem = rest[:_K], rest[_K]
    wid = lax.axis_index("s") * _NC + lax.axis_index("c")
    base = wid * rpw
    pltpu.sync_copy(idx_hbm.at[pl.ds(base, rpw)], idx_v)
    _gather_loop(table_hbm, out_hbm, idx_v, bufs, sem, rpw, base)


def _mesh():
    return plsc.VectorSubcoreMesh(
        core_axis_name="c", subcore_axis_name="s",
        num_cores=_NC, num_subcores=_NS)


@functools.cache
def _make_gather_st(R, NP):
    rpw = R // _NW
    srpw = NP // _NW
    return pl.kernel(
        functools.partial(_gather_body, rpw, srpw),
        out_type=(jax.ShapeDtypeStruct((R, 128), jnp.int32),
                  jax.ShapeDtypeStruct((NP, S), jnp.float32)),
        mesh=_mesh(),
        scratch_types=[
            pltpu.VMEM((rpw,), jnp.int32),
            pltpu.VMEM((srpw,), jnp.int32),
            pltpu.VMEM((_CHUNK, S), jnp.float32),
        ] + [pltpu.VMEM((_CHUNK, 128), jnp.int32) for _ in range(_K)]
          + [pltpu.SemaphoreType.DMA],
    )


@functools.cache
def _make_gather(R):
    rpw = R // _NW
    return pl.kernel(
        functools.partial(_gather2_body, rpw),
        out_type=jax.ShapeDtypeStruct((R, 128), jnp.int32),
        mesh=_mesh(),
        scratch_types=[
            pltpu.VMEM((rpw,), jnp.int32),
        ] + [pltpu.VMEM((_CHUNK, 128), jnp.int32) for _ in range(_K)]
          + [pltpu.SemaphoreType.DMA],
    )


def _pack_rows(x):
    bits = lax.bitcast_convert_type(x, jnp.uint32)
    lo = (bits[:, :128] + jnp.uint32(0x8000)) >> 16
    hi = (bits[:, 128:] + jnp.uint32(0x8000)) & jnp.uint32(0xFFFF0000)
    return lax.bitcast_convert_type(hi | lo, jnp.int32)


def _unpack_rows(p):
    bits = lax.bitcast_convert_type(p, jnp.uint32)
    lo = lax.bitcast_convert_type(bits << 16, jnp.float32)
    hi = lax.bitcast_convert_type(bits & jnp.uint32(0xFFFF0000), jnp.float32)
    return lo.astype(jnp.bfloat16), hi.astype(jnp.bfloat16)


def _pack_stage(x_ref, out_ref):
    out_ref[...] = _pack_rows(x_ref[...])


def _atom_stage(a_ref, g_ref, nbr_ref, st_ref, wself, wnbr, wbond, wst, ba_ref,
                out_ref, pk_ref):
    a = a_ref[...]
    base = (jnp.dot(a, wself[...], preferred_element_type=jnp.float32)
            + jnp.dot(st_ref[...], wst[...], preferred_element_type=jnp.float32)
            + ba_ref[...])
    glo, ghi = _unpack_rows(g_ref[...].reshape(BN * M, 128))
    nb = nbr_ref[...].reshape(BN * M, NB)
    w = wnbr[...]
    zz = (jnp.dot(glo, w[:128], preferred_element_type=jnp.float32)
          + jnp.dot(ghi, w[128:], preferred_element_type=jnp.float32)
          + jnp.dot(nb, wbond[...], preferred_element_type=jnp.float32))
    z = zz.reshape(BN, M, 2 * A) + base[:, None, :]
    filt = z[..., :A]
    core = z[..., A:]
    acc = jnp.sum(_sigmoid(filt) * _softplus(core), axis=1)
    o = _softplus(a + acc)
    out_ref[...] = o
    pk_ref[...] = _pack_rows(o)


def _bond_stage(nbr_ref, g_ref, ao_ref, st_ref, idx_ref, wself, wnbr, wbond,
                wst, bb_ref, nbr_out_ref, pools_ref, acc_a, acc_b, acc_c):
    i = pl.program_id(0)

    @pl.when(i == 0)
    def _():
        acc_a[...] = jnp.zeros_like(acc_a)
        acc_b[...] = jnp.zeros_like(acc_b)
        acc_c[...] = jnp.zeros_like(acc_c)

    ao = ao_ref[...]
    ub = (jnp.dot(ao, wself[...], preferred_element_type=jnp.float32)
          + jnp.dot(st_ref[...], wst[...], preferred_element_type=jnp.float32)
          + bb_ref[...])
    glo, ghi = _unpack_rows(g_ref[...].reshape(BN * M, 128))
    nb3 = nbr_ref[...]
    nb = nb3.reshape(BN * M, NB).astype(jnp.bfloat16)
    w = wnbr[...]
    zz = (jnp.dot(glo, w[:128], preferred_element_type=jnp.float32)
          + jnp.dot(ghi, w[128:], preferred_element_type=jnp.float32)
          + jnp.dot(nb, wbond[...], preferred_element_type=jnp.float32))
    z = zz.reshape(BN, M, 2 * NB) + ub[:, None, :]
    filt = z[..., :NB]
    core = z[..., NB:]
    no = _softplus(nb3 + _sigmoid(filt) * _softplus(core))
    nbr_out_ref[...] = no

    bm = jnp.mean(no, axis=1)
    idx_t = idx_ref[...].reshape(1, BN)
    oh = (lax.broadcasted_iota(jnp.int32, (B, BN), 0) == idx_t
          ).astype(jnp.float32)
    acc_a[...] = acc_a[...] + jnp.dot(oh, ao, preferred_element_type=jnp.float32)
    acc_b[...] = acc_b[...] + jnp.dot(oh, bm, preferred_element_type=jnp.float32)
    acc_c[...] = acc_c[...] + jnp.sum(oh, axis=1, keepdims=True)

    @pl.when(i == GRIDC - 1)
    def _():
        pools_ref[...] = jnp.concatenate(
            [acc_a[...], acc_b[...], acc_c[...]], axis=-1)


def _state_stage(pools_ref, sf_ref, ws_ref, bs_ref, out_ref):
    p = pools_ref[...]
    cnt = jnp.maximum(p[..., A + NB:], 1.0)
    cnt_a = jnp.concatenate([cnt, cnt], axis=-1)
    ap = p[..., :A] / cnt_a
    bp = p[..., A:A + NB] / cnt
    sf = sf_ref[...]
    t3 = jnp.concatenate([ap, bp, sf], axis=-1)
    out_ref[...] = _softplus(
        sf + jnp.dot(t3, ws_ref[...], preferred_element_type=jnp.float32)
        + bs_ref[...])


def _full(shape):
    nd = len(shape)
    return pl.BlockSpec(shape, lambda i: (0,) * nd)


def kernel(atom_fea, nbr_fea, state_fea, Wa, ba, Wb, bb, Ws, bs,
           atom_nbr_idx, node_atom_idx):
    flat_idx = atom_nbr_idx.reshape(-1).astype(jnp.int32)
    node_idx = node_atom_idx.astype(jnp.int32)

    wa_self, wa_nbr = Wa[:A], Wa[A:2 * A].astype(jnp.bfloat16)
    wa_bond, wa_st = Wa[2 * A:2 * A + NB].astype(jnp.bfloat16), Wa[2 * A + NB:]
    wb_self, wb_nbr = Wb[:A], Wb[A:2 * A].astype(jnp.bfloat16)
    wb_bond, wb_st = Wb[2 * A:2 * A + NB].astype(jnp.bfloat16), Wb[2 * A + NB:]
    ba2 = ba.reshape(1, 2 * A)
    bb2 = bb.reshape(1, 2 * NB)
    bs2 = bs.reshape(1, S)
    nbr_bf = nbr_fea.astype(jnp.bfloat16)

    gather_st = _make_gather_st(N * M, NPC)
    gather = _make_gather(N * M)
    node_idx3 = node_idx.reshape(GRIDC, 1, BN)

    table1 = pl.pallas_call(
        _pack_stage,
        grid=(GRIDC,),
        in_specs=[pl.BlockSpec((BN, A), lambda i: (i, 0))],
        out_specs=pl.BlockSpec((BN, 128), lambda i: (i, 0)),
        out_shape=jax.ShapeDtypeStruct((N, 128), jnp.int32),
    )(atom_fea)

    sidx = jnp.concatenate([node_idx, jnp.zeros((NPC - N,), jnp.int32)])
    g1, st_full = gather_st(table1, flat_idx, state_fea, sidx)
    g1 = g1.reshape(N, M, 128)
    st = st_full[:N]

    atom_out, table2 = pl.pallas_call(
        _atom_stage,
        grid=(GRIDC,),
        in_specs=[
            pl.BlockSpec((BN, A), lambda i: (i, 0)),
            pl.BlockSpec((BN, M, 128), lambda i: (i, 0, 0)),
            pl.BlockSpec((BN, M, NB), lambda i: (i, 0, 0)),
            pl.BlockSpec((BN, S), lambda i: (i, 0)),
            _full((A, 2 * A)),
            _full((A, 2 * A)),
            _full((NB, 2 * A)),
            _full((S, 2 * A)),
            _full((1, 2 * A)),
        ],
        out_specs=[
            pl.BlockSpec((BN, A), lambda i: (i, 0)),
            pl.BlockSpec((BN, 128), lambda i: (i, 0)),
        ],
        out_shape=[
            jax.ShapeDtypeStruct((N, A), jnp.float32),
            jax.ShapeDtypeStruct((N, 128), jnp.int32),
        ],
    )(atom_fea, g1, nbr_bf, st, wa_self, wa_nbr, wa_bond, wa_st, ba2)

    g2 = gather(table2, flat_idx).reshape(N, M, 128)

    nbr_out, pools = pl.pallas_call(
        _bond_stage,
        grid=(GRIDC,),
        in_specs=[
            pl.BlockSpec((BN, M, NB), lambda i: (i, 0, 0)),
            pl.BlockSpec((BN, M, 128), lambda i: (i, 0, 0)),
            pl.BlockSpec((BN, A), lambda i: (i, 0)),
            pl.BlockSpec((BN, S), lambda i: (i, 0)),
            pl.BlockSpec((1, 1, BN), lambda i: (i, 0, 0)),
            _full((A, 2 * NB)),
            _full((A, 2 * NB)),
            _full((NB, 2 * NB)),
            _full((S, 2 * NB)),
            _full((1, 2 * NB)),
        ],
        out_specs=[
            pl.BlockSpec((BN, M, NB), lambda i: (i, 0, 0)),
            pl.BlockSpec((B, A + 2 * NB), lambda i: (0, 0)),
        ],
        out_shape=[
            jax.ShapeDtypeStruct((N, M, NB), jnp.float32),
            jax.ShapeDtypeStruct((B, A + 2 * NB), jnp.float32),
        ],
        scratch_shapes=[
            pltpu.VMEM((B, A), jnp.float32),
            pltpu.VMEM((B, NB), jnp.float32),
            pltpu.VMEM((B, NB), jnp.float32),
        ],
    )(nbr_fea, g2, atom_out, st, node_idx3,
      wb_self, wb_nbr, wb_bond, wb_st, bb2)

    state_out = pl.pallas_call(
        _state_stage,
        grid=(1,),
        in_specs=[
            _full((B, A + 2 * NB)),
            _full((B, S)),
            _full((A + NB + S, S)),
            _full((1, S)),
        ],
        out_specs=pl.BlockSpec((B, S), lambda i: (0, 0)),
        out_shape=jax.ShapeDtypeStruct((B, S), jnp.float32),
    )(pools, state_fea, Ws, bs2)

    return atom_out, nbr_out, state_out


# speedup vs baseline: 7.6799x; 1.1537x over previous
import functools

import jax
import jax.numpy as jnp
from jax import lax
from jax.experimental import pallas as pl
from jax.experimental.pallas import tpu as pltpu
from jax.experimental.pallas import tpu_sc as plsc

N = 10000
M = 16
A = 256
NB = 128
S = 128
B = 128

BN = 200
GRIDC = N // BN

_NC = 2
_NS = 16
_NW = _NC * _NS


def _sigmoid(x):
    return 0.5 * jnp.tanh(0.5 * x) + 0.5


def _softplus(x):
    return jnp.maximum(x, 0.0) + jnp.log1p(jnp.exp(-jnp.abs(x)))


_CHUNK = 40
_K = 5


def _gather_body(rpw, table_hbm, idx_hbm, out_hbm, idx_v, *rest):
    bufs, sem = rest[:_K], rest[_K]
    wid = lax.axis_index("s") * _NC + lax.axis_index("c")
    base = wid * rpw
    pltpu.sync_copy(idx_hbm.at[pl.ds(base, rpw)], idx_v)
    ngroups = rpw // (_K * _CHUNK)

    def group(g, carry):
        goff = g * (_K * _CHUNK)
        copies = []
        for j in range(_K):
            copies.append(pltpu.async_copy(
                table_hbm.at[idx_v.at[pl.ds(goff + j * _CHUNK, _CHUNK)]],
                bufs[j], sem))
        for j in range(_K):
            copies[j].wait()
            pltpu.sync_copy(
                bufs[j], out_hbm.at[pl.ds(base + goff + j * _CHUNK, _CHUNK)])
        return carry

    lax.fori_loop(0, ngroups, group, 0)


def _mesh():
    return plsc.VectorSubcoreMesh(
        core_axis_name="c", subcore_axis_name="s",
        num_cores=_NC, num_subcores=_NS)


@functools.cache
def _make_gather(R):
    rpw = R // _NW
    return pl.kernel(
        functools.partial(_gather_body, rpw),
        out_type=jax.ShapeDtypeStruct((R, 128), jnp.int32),
        mesh=_mesh(),
        scratch_types=[
            pltpu.VMEM((rpw,), jnp.int32),
        ] + [pltpu.VMEM((_CHUNK, 128), jnp.int32) for _ in range(_K)]
          + [pltpu.SemaphoreType.DMA],
    )


def _pack_rows(x):
    bits = lax.bitcast_convert_type(x, jnp.uint32)
    lo = (bits[:, :128] + jnp.uint32(0x8000)) >> 16
    hi = (bits[:, 128:] + jnp.uint32(0x8000)) & jnp.uint32(0xFFFF0000)
    return lax.bitcast_convert_type(hi | lo, jnp.int32)


def _unpack_rows(p):
    bits = lax.bitcast_convert_type(p, jnp.uint32)
    lo = lax.bitcast_convert_type(bits << 16, jnp.float32)
    hi = lax.bitcast_convert_type(bits & jnp.uint32(0xFFFF0000), jnp.float32)
    return lo.astype(jnp.bfloat16), hi.astype(jnp.bfloat16)


def _pack_stage(x_ref, out_ref):
    out_ref[...] = _pack_rows(x_ref[...])


def _one_hot_t(idxc):
    return (lax.broadcasted_iota(jnp.int32, (BN, B), 1) == idxc
            ).astype(jnp.float32)


def _atom_stage(a_ref, g_ref, nbr_ref, idxc_ref, sf_ref, wself, wcat, wst,
                ba_ref, out_ref, pk_ref):
    a = a_ref[...]
    sfw = jnp.dot(sf_ref[...], wst[...], preferred_element_type=jnp.float32)
    oht = _one_hot_t(idxc_ref[...])
    base = (jnp.dot(a, wself[...], preferred_element_type=jnp.float32)
            + jnp.dot(oht, sfw, preferred_element_type=jnp.float32)
            + ba_ref[...])
    glo, ghi = _unpack_rows(g_ref[...].reshape(BN * M, 128))
    nb = nbr_ref[...].reshape(BN * M, NB)
    x = jnp.concatenate([glo, ghi, nb], axis=1)
    zz = jnp.dot(x, wcat[...], preferred_element_type=jnp.float32)
    z = zz.reshape(BN, M, 2 * A) + base[:, None, :]
    filt = z[..., :A]
    core = z[..., A:]
    acc = jnp.sum(_sigmoid(filt) * _softplus(core), axis=1)
    o = _softplus(a + acc)
    out_ref[...] = o
    pk_ref[...] = _pack_rows(o)


def _bond_stage(nbr_ref, g_ref, ao_ref, idxc_ref, idx_ref, sf_ref, wself,
                wcat, wst, bb_ref, ws_ref, bs_ref, nbr_out_ref, st_out_ref,
                acc_a, acc_b, acc_c):
    i = pl.program_id(0)

    @pl.when(i == 0)
    def _():
        acc_a[...] = jnp.zeros_like(acc_a)
        acc_b[...] = jnp.zeros_like(acc_b)
        acc_c[...] = jnp.zeros_like(acc_c)

    ao = ao_ref[...]
    sf = sf_ref[...]
    sfw = jnp.dot(sf, wst[...], preferred_element_type=jnp.float32)
    oht = _one_hot_t(idxc_ref[...])
    ub = (jnp.dot(ao, wself[...], preferred_element_type=jnp.float32)
          + jnp.dot(oht, sfw, preferred_element_type=jnp.float32)
          + bb_ref[...])
    glo, ghi = _unpack_rows(g_ref[...].reshape(BN * M, 128))
    nb3 = nbr_ref[...]
    nb = nb3.reshape(BN * M, NB).astype(jnp.bfloat16)
    x = jnp.concatenate([glo, ghi, nb], axis=1)
    zz = jnp.dot(x, wcat[...], preferred_element_type=jnp.float32)
    z = zz.reshape(BN, M, 2 * NB) + ub[:, None, :]
    filt = z[..., :NB]
    core = z[..., NB:]
    no = _softplus(nb3 + _sigmoid(filt) * _softplus(core))
    nbr_out_ref[...] = no

    bm = jnp.mean(no, axis=1)
    idx_t = idx_ref[...].reshape(1, BN)
    oh = (lax.broadcasted_iota(jnp.int32, (B, BN), 0) == idx_t
          ).astype(jnp.float32)
    acc_a[...] = acc_a[...] + jnp.dot(oh, ao, preferred_element_type=jnp.float32)
    acc_b[...] = acc_b[...] + jnp.dot(oh, bm, preferred_element_type=jnp.float32)
    acc_c[...] = acc_c[...] + jnp.sum(oh, axis=1, keepdims=True)

    @pl.when(i == GRIDC - 1)
    def _():
        cnt = jnp.maximum(acc_c[...], 1.0)
        cnt_a = jnp.concatenate([cnt, cnt], axis=-1)
        ap = acc_a[...] / cnt_a
        bp = acc_b[...] / cnt
        t3 = jnp.concatenate([ap, bp, sf], axis=-1)
        st_out_ref[...] = _softplus(
            sf + jnp.dot(t3, ws_ref[...], preferred_element_type=jnp.float32)
            + bs_ref[...])


def _full(shape):
    nd = len(shape)
    return pl.BlockSpec(shape, lambda i: (0,) * nd)


def kernel(atom_fea, nbr_fea, state_fea, Wa, ba, Wb, bb, Ws, bs,
           atom_nbr_idx, node_atom_idx):
    flat_idx = atom_nbr_idx.reshape(-1).astype(jnp.int32)
    node_idx = node_atom_idx.astype(jnp.int32)

    wa_self, wa_nbr = Wa[:A], Wa[A:2 * A].astype(jnp.bfloat16)
    wa_bond, wa_st = Wa[2 * A:2 * A + NB].astype(jnp.bfloat16), Wa[2 * A + NB:]
    wb_self, wb_nbr = Wb[:A], Wb[A:2 * A].astype(jnp.bfloat16)
    wb_bond, wb_st = Wb[2 * A:2 * A + NB].astype(jnp.bfloat16), Wb[2 * A + NB:]
    wa_cat = jnp.concatenate([wa_nbr, wa_bond], axis=0)
    wb_cat = jnp.concatenate([wb_nbr, wb_bond], axis=0)
    ba2 = ba.reshape(1, 2 * A)
    bb2 = bb.reshape(1, 2 * NB)
    bs2 = bs.reshape(1, S)
    nbr_bf = nbr_fea.astype(jnp.bfloat16)
    node_idx3 = node_idx.reshape(GRIDC, 1, BN)
    node_idxc = node_idx.reshape(N, 1)

    gather = _make_gather(N * M)

    table1 = pl.pallas_call(
        _pack_stage,
        grid=(GRIDC,),
        in_specs=[pl.BlockSpec((BN, A), lambda i: (i, 0))],
        out_specs=pl.BlockSpec((BN, 128), lambda i: (i, 0)),
        out_shape=jax.ShapeDtypeStruct((N, 128), jnp.int32),
    )(atom_fea)

    g1 = gather(table1, flat_idx).reshape(N, M, 128)

    atom_out, table2 = pl.pallas_call(
        _atom_stage,
        grid=(GRIDC,),
        in_specs=[
            pl.BlockSpec((BN, A), lambda i: (i, 0)),
            pl.BlockSpec((BN, M, 128), lambda i: (i, 0, 0)),
            pl.BlockSpec((BN, M, NB), lambda i: (i, 0, 0)),
            pl.BlockSpec((BN, 1), lambda i: (i, 0)),
            _full((B, S)),
            _full((A, 2 * A)),
            _full((A + NB, 2 * A)),
            _full((S, 2 * A)),
            _full((1, 2 * A)),
        ],
        out_specs=[
            pl.BlockSpec((BN, A), lambda i: (i, 0)),
            pl.BlockSpec((BN, 128), lambda i: (i, 0)),
        ],
        out_shape=[
            jax.ShapeDtypeStruct((N, A), jnp.float32),
            jax.ShapeDtypeStruct((N, 128), jnp.int32),
        ],
    )(atom_fea, g1, nbr_bf, node_idxc, state_fea,
      wa_self, wa_cat, wa_st, ba2)

    g2 = gather(table2, flat_idx).reshape(N, M, 128)

    nbr_out, state_out = pl.pallas_call(
        _bond_stage,
        grid=(GRIDC,),
        in_specs=[
            pl.BlockSpec((BN, M, NB), lambda i: (i, 0, 0)),
            pl.BlockSpec((BN, M, 128), lambda i: (i, 0, 0)),
            pl.BlockSpec((BN, A), lambda i: (i, 0)),
            pl.BlockSpec((BN, 1), lambda i: (i, 0)),
            pl.BlockSpec((1, 1, BN), lambda i: (i, 0, 0)),
            _full((B, S)),
            _full((A, 2 * NB)),
            _full((A + NB, 2 * NB)),
            _full((S, 2 * NB)),
            _full((1, 2 * NB)),
            _full((A + NB + S, S)),
            _full((1, S)),
        ],
        out_specs=[
            pl.BlockSpec((BN, M, NB), lambda i: (i, 0, 0)),
            pl.BlockSpec((B, S), lambda i: (0, 0)),
        ],
        out_shape=[
            jax.ShapeDtypeStruct((N, M, NB), jnp.float32),
            jax.ShapeDtypeStruct((B, S), jnp.float32),
        ],
        scratch_shapes=[
            pltpu.VMEM((B, A), jnp.float32),
            pltpu.VMEM((B, NB), jnp.float32),
            pltpu.VMEM((B, NB), jnp.float32),
        ],
    )(nbr_fea, g2, atom_out, node_idxc, node_idx3, state_fea,
      wb_self, wb_cat, wb_st, bb2, Ws, bs2)

    return atom_out, nbr_out, state_out
